# baseline (speedup 1.0000x reference)
import dataclasses
import functools

import jax
import jax.numpy as jnp
from jax import lax
from jax.experimental import pallas as pl
from jax.experimental.pallas import tpu as pltpu
from jax.experimental.pallas import tpu_sc as plsc

_NC = 2
_NS = 16
_NW = _NC * _NS


def _embed_sc(idx, betas, go, bp, tr):
    B = idx.shape[0]
    b_per_w = B // _NW
    d_be, d_go, d_bp, d_tr = betas.shape[1], go.shape[1], bp.shape[1], tr.shape[1]
    mesh = plsc.VectorSubcoreMesh(core_axis_name="c", subcore_axis_name="s")
    cp = pltpu.CompilerParams()
    if "needs_layout_passes" in pltpu.CompilerParams.__dataclass_fields__:
        cp = dataclasses.replace(cp, needs_layout_passes=False)

    @functools.partial(
        pl.kernel,
        mesh=mesh,
        compiler_params=cp,
        out_type=(
            jax.ShapeDtypeStruct((B, d_be), betas.dtype),
            jax.ShapeDtypeStruct((B, d_go), go.dtype),
            jax.ShapeDtypeStruct((B, d_bp), bp.dtype),
            jax.ShapeDtypeStruct((B, d_tr), tr.dtype),
        ),
        scratch_types=[
            pltpu.VMEM((b_per_w,), jnp.int32),
            pltpu.VMEM((1, d_be), betas.dtype),
            pltpu.VMEM((b_per_w, d_be), betas.dtype),
            pltpu.VMEM((b_per_w, d_go), go.dtype),
            pltpu.VMEM((b_per_w, d_bp), bp.dtype),
            pltpu.VMEM((b_per_w, d_tr), tr.dtype),
            pltpu.SemaphoreType.DMA,
        ],
    )
    def k(be_hbm, go_hbm, bp_hbm, tr_hbm, idx_hbm, obe_hbm, ogo_hbm, obp_hbm, otr_hbm,
          idx_v, bsrc_v, be_v, go_v, bp_v, tr_v, sem):
        wid = lax.axis_index("s") * _NC + lax.axis_index("c")
        base = wid * b_per_w
        pltpu.sync_copy(idx_hbm.at[pl.ds(base, b_per_w)], idx_v)

        pltpu.sync_copy(be_hbm.at[pl.ds(0, 1)], bsrc_v)
        zeros16 = lax.iota(jnp.int32, 16) * 0

        @pl.loop(0, b_per_w * d_be, step=16)
        def _(off0):
            off = off0 + lax.iota(jnp.int32, 16)
            r = off // d_be
            c = off - r * d_be
            vals = plsc.load_gather(bsrc_v, [zeros16, c])
            plsc.store_scatter(be_v, [r, c], vals)

        pltpu.sync_copy(be_v, obe_hbm.at[pl.ds(base, b_per_w)])

        @pl.loop(0, b_per_w, step=16)
        def _(c):
            v = idx_v[pl.ds(c, 16)]
            for k in range(16):
                j = v[k]
                pltpu.async_copy(go_hbm.at[pl.ds(j, 1)], go_v.at[pl.ds(c + k, 1)], sem)
                pltpu.async_copy(bp_hbm.at[pl.ds(j, 1)], bp_v.at[pl.ds(c + k, 1)], sem)
                pltpu.async_copy(tr_hbm.at[pl.ds(j, 1)], tr_v.at[pl.ds(c + k, 1)], sem)

        @pl.loop(0, b_per_w)
        def _(i):
            pltpu.make_async_copy(go_hbm.at[pl.ds(0, 1)], go_v.at[pl.ds(i, 1)], sem).wait()
            pltpu.make_async_copy(bp_hbm.at[pl.ds(0, 1)], bp_v.at[pl.ds(i, 1)], sem).wait()
            pltpu.make_async_copy(tr_hbm.at[pl.ds(0, 1)], tr_v.at[pl.ds(i, 1)], sem).wait()

        pltpu.sync_copy(go_v, ogo_hbm.at[pl.ds(base, b_per_w)])
        pltpu.sync_copy(bp_v, obp_hbm.at[pl.ds(base, b_per_w)])
        pltpu.sync_copy(tr_v, otr_hbm.at[pl.ds(base, b_per_w)])

    return k(betas, go, bp, tr, idx)


def kernel(idx, betas, global_orient, body_pose, transl):
    idx = idx.astype(jnp.int32)
    return _embed_sc(idx, betas, global_orient, body_pose, transl)


# speedup vs baseline: 1.0032x; 1.0032x over previous
import dataclasses
import functools

import jax
import jax.numpy as jnp
from jax import lax
from jax.experimental import pallas as pl
from jax.experimental.pallas import tpu as pltpu
from jax.experimental.pallas import tpu_sc as plsc

_NC = 2
_NS = 16
_NW = _NC * _NS


def _embed_sc(idx, betas, go, bp, tr):
    B = idx.shape[0]
    b_per_w = B // _NW
    d_be, d_go, d_bp, d_tr = betas.shape[1], go.shape[1], bp.shape[1], tr.shape[1]
    mesh = plsc.VectorSubcoreMesh(core_axis_name="c", subcore_axis_name="s")
    cp = pltpu.CompilerParams()
    if "needs_layout_passes" in pltpu.CompilerParams.__dataclass_fields__:
        cp = dataclasses.replace(cp, needs_layout_passes=False)
    if "use_tc_tiling_on_sc" in pltpu.CompilerParams.__dataclass_fields__:
        cp = dataclasses.replace(cp, use_tc_tiling_on_sc=True)

    @functools.partial(
        pl.kernel,
        mesh=mesh,
        compiler_params=cp,
        out_type=(
            jax.ShapeDtypeStruct((B, d_be), betas.dtype),
            jax.ShapeDtypeStruct((B, d_go), go.dtype),
            jax.ShapeDtypeStruct((B, d_bp), bp.dtype),
            jax.ShapeDtypeStruct((B, d_tr), tr.dtype),
        ),
        scratch_types=[
            pltpu.VMEM((b_per_w,), jnp.int32),
            pltpu.VMEM((1, d_be), betas.dtype),
            pltpu.VMEM((b_per_w, d_be), betas.dtype),
            pltpu.VMEM((b_per_w, d_go), go.dtype),
            pltpu.VMEM((b_per_w, d_bp), bp.dtype),
            pltpu.VMEM((b_per_w, d_tr), tr.dtype),
            pltpu.SemaphoreType.DMA,
        ],
    )
    def k(be_hbm, go_hbm, bp_hbm, tr_hbm, idx_hbm, obe_hbm, ogo_hbm, obp_hbm, otr_hbm,
          idx_v, bsrc_v, be_v, go_v, bp_v, tr_v, sem):
        wid = lax.axis_index("s") * _NC + lax.axis_index("c")
        base = wid * b_per_w
        pltpu.sync_copy(idx_hbm.at[pl.ds(base, b_per_w)], idx_v)

        pltpu.sync_copy(be_hbm.at[pl.ds(0, 1)], bsrc_v)
        zeros16 = lax.iota(jnp.int32, 16) * 0

        @pl.loop(0, b_per_w * d_be, step=16)
        def _(off0):
            off = off0 + lax.iota(jnp.int32, 16)
            r = off // d_be
            c = off - r * d_be
            vals = plsc.load_gather(bsrc_v, [zeros16, c])
            plsc.store_scatter(be_v, [r, c], vals)

        pltpu.sync_copy(be_v, obe_hbm.at[pl.ds(base, b_per_w)])

        @pl.loop(0, b_per_w, step=16)
        def _(c):
            v = idx_v[pl.ds(c, 16)]
            for k in range(16):
                j = v[k]
                pltpu.async_copy(go_hbm.at[pl.ds(j, 1)], go_v.at[pl.ds(c + k, 1)], sem)
                pltpu.async_copy(bp_hbm.at[pl.ds(j, 1)], bp_v.at[pl.ds(c + k, 1)], sem)
                pltpu.async_copy(tr_hbm.at[pl.ds(j, 1)], tr_v.at[pl.ds(c + k, 1)], sem)

        @pl.loop(0, b_per_w)
        def _(i):
            pltpu.make_async_copy(go_hbm.at[pl.ds(0, 1)], go_v.at[pl.ds(i, 1)], sem).wait()
            pltpu.make_async_copy(bp_hbm.at[pl.ds(0, 1)], bp_v.at[pl.ds(i, 1)], sem).wait()
            pltpu.make_async_copy(tr_hbm.at[pl.ds(0, 1)], tr_v.at[pl.ds(i, 1)], sem).wait()

        pltpu.sync_copy(go_v, ogo_hbm.at[pl.ds(base, b_per_w)])
        pltpu.sync_copy(bp_v, obp_hbm.at[pl.ds(base, b_per_w)])
        pltpu.sync_copy(tr_v, otr_hbm.at[pl.ds(base, b_per_w)])

    return k(betas, go, bp, tr, idx)


def kernel(idx, betas, global_orient, body_pose, transl):
    idx = idx.astype(jnp.int32)
    return _embed_sc(idx, betas, global_orient, body_pose, transl)
